# Initial kernel scaffold; baseline (speedup 1.0000x reference)
import jax
import jax.numpy as jnp
from jax.experimental import pallas as pl


def kernel(position_ids, h_table, w_table):
    raise NotImplementedError("write your pallas kernel here")


# speedup vs baseline: 3.6763x; 3.6763x over previous
import functools

import jax
import jax.numpy as jnp
from jax import lax
from jax.experimental import pallas as pl
from jax.experimental.pallas import tpu as pltpu
from jax.experimental.pallas import tpu_sc as plsc

DIM = 768
BATCH = 64
SEQ = 576
ROWS = BATCH * SEQ
TAB = 64

NUM_CORES = 2
NUM_SUBCORES = 16
NUM_WORKERS = NUM_CORES * NUM_SUBCORES
ROWS_PER_WORKER = ROWS // NUM_WORKERS
LANES = 16
IDX_STEPS = ROWS_PER_WORKER // LANES
CHUNK = 64
NUM_CHUNKS = ROWS_PER_WORKER // CHUNK


def _sum_table_tc(h_ref, w_ref, out_ref):
    h = h_ref[...]
    w = w_ref[...]
    out_ref[...] = h[:, None, :] + w[None, :, :]


def _build_sum_table(h_table, w_table):
    return pl.pallas_call(
        _sum_table_tc,
        out_shape=jax.ShapeDtypeStruct((TAB, TAB, DIM), jnp.float32),
    )(h_table, w_table)


_MESH = plsc.VectorSubcoreMesh(core_axis_name="c", subcore_axis_name="s")


@functools.partial(
    pl.kernel,
    out_type=jax.ShapeDtypeStruct((ROWS, DIM), jnp.float32),
    mesh=_MESH,
    scratch_types=[
        pltpu.VMEM((ROWS_PER_WORKER,), jnp.int32),
        pltpu.VMEM((ROWS_PER_WORKER,), jnp.int32),
        pltpu.VMEM((CHUNK, DIM), jnp.float32),
        pltpu.SemaphoreType.DMA,
    ],
)
def _gather_sc(h_idx_hbm, w_idx_hbm, sum_tab_hbm, out_hbm,
               hidx_v, widx_v, buf_v, sem):
    wid = lax.axis_index("s") * NUM_CORES + lax.axis_index("c")
    base = wid * ROWS_PER_WORKER
    pltpu.sync_copy(h_idx_hbm.at[pl.ds(base, ROWS_PER_WORKER)], hidx_v)
    pltpu.sync_copy(w_idx_hbm.at[pl.ds(base, ROWS_PER_WORKER)], widx_v)

    def fuse_body(k, carry):
        sl = pl.ds(k * LANES, LANES)
        hidx_v[sl] = hidx_v[sl] * TAB + widx_v[sl]
        return carry

    lax.fori_loop(0, IDX_STEPS, fuse_body, 0)

    def chunk_body(i, carry):
        off = i * CHUNK
        pltpu.async_copy(
            sum_tab_hbm.at[hidx_v.at[pl.ds(off, CHUNK)]], buf_v, sem).wait()
        pltpu.sync_copy(buf_v, out_hbm.at[pl.ds(base + off, CHUNK)])
        return carry

    lax.fori_loop(0, NUM_CHUNKS, chunk_body, 0)


def kernel(position_ids, h_table, w_table):
    h_idx = position_ids[..., 0].reshape(ROWS).astype(jnp.int32)
    w_idx = position_ids[..., 1].reshape(ROWS).astype(jnp.int32)
    sum_tab = _build_sum_table(h_table, w_table).reshape(TAB * TAB, DIM)
    out = _gather_sc(h_idx, w_idx, sum_tab)
    return out.reshape(BATCH, SEQ, DIM)


# speedup vs baseline: 4.1005x; 1.1154x over previous
import functools

import jax
import jax.numpy as jnp
from jax import lax
from jax.experimental import pallas as pl
from jax.experimental.pallas import tpu as pltpu
from jax.experimental.pallas import tpu_sc as plsc

DIM = 768
BATCH = 64
SEQ = 576
ROWS = BATCH * SEQ
TAB = 64

NUM_CORES = 2
NUM_SUBCORES = 16
NUM_WORKERS = NUM_CORES * NUM_SUBCORES
ROWS_PER_WORKER = ROWS // NUM_WORKERS
LANES = 16
IDX_STEPS = ROWS_PER_WORKER // LANES
CHUNK = 64
NUM_CHUNKS = ROWS_PER_WORKER // CHUNK
NUM_PAIRS = NUM_CHUNKS // 2


def _sum_table_tc(h_ref, w_ref, out_ref):
    h = h_ref[...]
    w = w_ref[...]
    out_ref[...] = h[:, None, :] + w[None, :, :]


def _build_sum_table(h_table, w_table):
    return pl.pallas_call(
        _sum_table_tc,
        out_shape=jax.ShapeDtypeStruct((TAB, TAB, DIM), jnp.float32),
    )(h_table, w_table)


_MESH = plsc.VectorSubcoreMesh(core_axis_name="c", subcore_axis_name="s")


@functools.partial(
    pl.kernel,
    out_type=jax.ShapeDtypeStruct((ROWS, DIM), jnp.float32),
    mesh=_MESH,
    scratch_types=[
        pltpu.VMEM((ROWS_PER_WORKER,), jnp.int32),
        pltpu.VMEM((ROWS_PER_WORKER,), jnp.int32),
        pltpu.VMEM((CHUNK, DIM), jnp.float32),
        pltpu.VMEM((CHUNK, DIM), jnp.float32),
        pltpu.SemaphoreType.DMA,
        pltpu.SemaphoreType.DMA,
    ],
)
def _gather_sc(h_idx_hbm, w_idx_hbm, sum_tab_hbm, out_hbm,
               hidx_v, widx_v, buf_a, buf_b, sem_a, sem_b):
    wid = lax.axis_index("s") * NUM_CORES + lax.axis_index("c")
    base = wid * ROWS_PER_WORKER
    pltpu.sync_copy(h_idx_hbm.at[pl.ds(base, ROWS_PER_WORKER)], hidx_v)
    pltpu.sync_copy(w_idx_hbm.at[pl.ds(base, ROWS_PER_WORKER)], widx_v)

    def fuse_body(k, carry):
        sl = pl.ds(k * LANES, LANES)
        hidx_v[sl] = hidx_v[sl] * TAB + widx_v[sl]
        return carry

    lax.fori_loop(0, IDX_STEPS, fuse_body, 0)

    def gather(chunk, buf, sem):
        return pltpu.async_copy(
            sum_tab_hbm.at[hidx_v.at[pl.ds(chunk * CHUNK, CHUNK)]], buf, sem)

    def wait_gather(chunk, buf, sem):
        pltpu.make_async_copy(
            sum_tab_hbm.at[hidx_v.at[pl.ds(chunk * CHUNK, CHUNK)]],
            buf, sem).wait()

    def writeback(chunk, buf):
        pltpu.sync_copy(buf, out_hbm.at[pl.ds(base + chunk * CHUNK, CHUNK)])

    gather(0, buf_a, sem_a)

    def pair_body(p, carry):
        g0 = 2 * p
        gather(g0 + 1, buf_b, sem_b)
        wait_gather(g0, buf_a, sem_a)
        writeback(g0, buf_a)

        @pl.when(p < NUM_PAIRS - 1)
        def _():
            gather(g0 + 2, buf_a, sem_a)

        wait_gather(g0 + 1, buf_b, sem_b)
        writeback(g0 + 1, buf_b)
        return carry

    lax.fori_loop(0, NUM_PAIRS, pair_body, 0)


def kernel(position_ids, h_table, w_table):
    h_idx = position_ids[..., 0].reshape(ROWS).astype(jnp.int32)
    w_idx = position_ids[..., 1].reshape(ROWS).astype(jnp.int32)
    sum_tab = _build_sum_table(h_table, w_table).reshape(TAB * TAB, DIM)
    out = _gather_sc(h_idx, w_idx, sum_tab)
    return out.reshape(BATCH, SEQ, DIM)
